# Initial kernel scaffold; baseline (speedup 1.0000x reference)
import jax
import jax.numpy as jnp
from jax import lax
from jax.experimental import pallas as pl
from jax.experimental.pallas import tpu as pltpu
from jax.experimental.pallas import tpu_sc as plsc

NC = 2
NS = 16
L = 16
NW = NC * NS
G = 313
OUT_ROWS = NW * G
D = 128
NV = D // L
B = 128
N = 320000
NBLK = N // B
NEG_INF = float("-inf")


def _seg_max_body(emb_hbm, ids_hbm, out_hbm,
                  acc, rowbuf, idbuf, probe_a, probe_b, sem_a, sem_b):
    cid = lax.axis_index("c")
    sid = lax.axis_index("s")
    wid = sid * NC + cid
    lo = wid * G

    ninf = jnp.full((L,), NEG_INF, jnp.float32)

    def init_row(j, carry):
        for v in range(NV):
            acc[j, pl.ds(v * L, L)] = ninf
        return carry

    lax.fori_loop(0, G + 2, init_row, 0)

    t1 = lo
    t2 = lo + G

    def bs_step(_, st):
        a1, b1, a2, b2 = st
        m1 = jnp.minimum((a1 + b1) // 2, NBLK - 1)
        m2 = jnp.minimum((a2 + b2) // 2, NBLK - 1)
        cp1 = pltpu.async_copy(ids_hbm.at[pl.ds(m1 * B, 8)], probe_a, sem_a)
        cp2 = pltpu.async_copy(ids_hbm.at[pl.ds(m2 * B, 8)], probe_b, sem_b)
        cp1.wait()
        cp2.wait()
        v1 = probe_a[0]
        v2 = probe_b[0]
        a1n = jnp.where(a1 < b1, jnp.where(v1 < t1, m1 + 1, a1), a1)
        b1n = jnp.where(a1 < b1, jnp.where(v1 < t1, b1, m1), b1)
        a2n = jnp.where(a2 < b2, jnp.where(v2 < t2, m2 + 1, a2), a2)
        b2n = jnp.where(a2 < b2, jnp.where(v2 < t2, b2, m2), b2)
        return (a1n, b1n, a2n, b2n)

    kk, _, ke, _ = lax.fori_loop(0, 12, bs_step, (0, NBLK, 0, NBLK))
    k0 = jnp.maximum(kk - 1, 0)
    nb = jnp.maximum(ke - k0, 0)

    def block_body(k, carry):
        r = (k0 + k) * B
        pltpu.sync_copy(ids_hbm.at[pl.ds(r, B)], idbuf)
        pltpu.sync_copy(emb_hbm.at[pl.ds(r, B), :], rowbuf)

        def row_body(i, c2):
            s = jnp.clip(idbuf[i] - lo, -1, G) + 1
            for v in range(NV):
                sl = pl.ds(v * L, L)
                acc[s, sl] = jnp.maximum(acc[s, sl], rowbuf[i, sl])
            return c2

        lax.fori_loop(0, B, row_body, 0)
        return carry

    lax.fori_loop(0, nb, block_body, 0)

    zero = jnp.zeros((L,), jnp.float32)

    def fix_row(j, carry):
        for v in range(NV):
            sl = pl.ds(v * L, L)
            x = acc[j + 1, sl]
            acc[j + 1, sl] = jnp.where(x == NEG_INF, zero, x)
        return carry

    lax.fori_loop(0, G, fix_row, 0)
    pltpu.sync_copy(acc.at[pl.ds(1, G)], out_hbm.at[pl.ds(lo, G)])


@jax.jit
def _seg_max(emb, ids32):
    mesh = plsc.VectorSubcoreMesh(core_axis_name="c", subcore_axis_name="s")
    f = pl.kernel(
        _seg_max_body,
        out_type=jax.ShapeDtypeStruct((OUT_ROWS, D), jnp.float32),
        mesh=mesh,
        scratch_types=[
            pltpu.VMEM((G + 2, D), jnp.float32),
            pltpu.VMEM((B, D), jnp.float32),
            pltpu.VMEM((B,), jnp.int32),
            pltpu.VMEM((8,), jnp.int32),
            pltpu.VMEM((8,), jnp.int32),
            pltpu.SemaphoreType.DMA,
            pltpu.SemaphoreType.DMA,
        ],
    )
    return f(emb, ids32)


def kernel(batch_gnn_embed, gather_idx, num_graph):
    ids32 = gather_idx.astype(jnp.int32)
    out = _seg_max(batch_gnn_embed, ids32)
    return out[:10000]


# speedup vs baseline: 1.4870x; 1.4870x over previous
import jax
import jax.numpy as jnp
from jax import lax
from jax.experimental import pallas as pl
from jax.experimental.pallas import tpu as pltpu
from jax.experimental.pallas import tpu_sc as plsc

NC = 2
NS = 16
L = 16
NW = NC * NS
G = 320
OUT_ROWS = NW * G
D = 128
NV = D // L
B = 128
N = 320000
NBLK = N // B
NEG_INF = float("-inf")


def _seg_max_body(emb_hbm, ids_hbm, out_hbm,
                  acc, rowbuf, idbuf, probe_a, probe_b, sem_a, sem_b):
    i32 = jnp.int32
    cid = lax.axis_index("c")
    sid = lax.axis_index("s")
    wid = (sid * NC + cid).astype(i32)
    lo = wid * i32(G)

    ninf = jnp.full((L,), NEG_INF, jnp.float32)

    def init_row(j, carry):
        for v in range(NV):
            acc[j, pl.ds(v * L, L)] = ninf
        return carry

    lax.fori_loop(i32(0), i32(G + 1), init_row, i32(0))

    t1 = lo
    t2 = lo + G

    def bs_step(_, st):
        a1, b1, a2, b2 = st
        m1 = jnp.minimum((a1 + b1) >> 1, i32(NBLK - 1))
        m2 = jnp.minimum((a2 + b2) >> 1, i32(NBLK - 1))
        cp1 = pltpu.async_copy(ids_hbm.at[pl.ds(m1 * B, L)], probe_a, sem_a)
        cp2 = pltpu.async_copy(ids_hbm.at[pl.ds(m2 * B, L)], probe_b, sem_b)
        cp1.wait()
        cp2.wait()
        v1 = probe_a[pl.ds(0, L)][0]
        v2 = probe_b[pl.ds(0, L)][0]
        a1n = jnp.where(a1 < b1, jnp.where(v1 < t1, m1 + 1, a1), a1)
        b1n = jnp.where(a1 < b1, jnp.where(v1 < t1, b1, m1), b1)
        a2n = jnp.where(a2 < b2, jnp.where(v2 < t2, m2 + 1, a2), a2)
        b2n = jnp.where(a2 < b2, jnp.where(v2 < t2, b2, m2), b2)
        return (a1n, b1n, a2n, b2n)

    kk, _, ke, _ = lax.fori_loop(
        i32(0), i32(12), bs_step, (i32(0), i32(NBLK), i32(0), i32(NBLK)))
    k0 = jnp.maximum(kk - i32(1), i32(0))
    nb = jnp.maximum(ke - k0, i32(0))

    def block_body(k, carry):
        r = (k0 + k) * i32(B)
        pltpu.sync_copy(ids_hbm.at[pl.ds(r, B)], idbuf)
        pltpu.sync_copy(emb_hbm.at[pl.ds(r, B), :], rowbuf)

        def group_body(g, c2):
            idvec = idbuf[pl.ds(g * i32(L), L)]
            slot = idvec - lo
            inr = (slot >= i32(0)) & (slot < i32(G))
            svec = jnp.where(inr, slot, i32(G))
            for j in range(L):
                s = svec[j]
                for v in range(NV):
                    sl = pl.ds(v * L, L)
                    acc[s, sl] = jnp.maximum(
                        acc[s, sl], rowbuf[g * i32(L) + i32(j), sl])
            return c2

        lax.fori_loop(i32(0), i32(B // L), group_body, i32(0))
        return carry

    lax.fori_loop(i32(0), nb, block_body, i32(0))

    zero = jnp.zeros((L,), jnp.float32)

    def fix_row(j, carry):
        for v in range(NV):
            sl = pl.ds(v * L, L)
            x = acc[j, sl]
            acc[j, sl] = jnp.where(x == NEG_INF, zero, x)
        return carry

    lax.fori_loop(i32(0), i32(G), fix_row, i32(0))
    pltpu.sync_copy(acc.at[pl.ds(0, G)], out_hbm.at[pl.ds(lo, G)])


@jax.jit
def _seg_max(emb, ids32):
    mesh = plsc.VectorSubcoreMesh(core_axis_name="c", subcore_axis_name="s")
    f = pl.kernel(
        _seg_max_body,
        out_type=jax.ShapeDtypeStruct((OUT_ROWS, D), jnp.float32),
        mesh=mesh,
        scratch_types=[
            pltpu.VMEM((G + 1, D), jnp.float32),
            pltpu.VMEM((B, D), jnp.float32),
            pltpu.VMEM((B,), jnp.int32),
            pltpu.VMEM((L,), jnp.int32),
            pltpu.VMEM((L,), jnp.int32),
            pltpu.SemaphoreType.DMA,
            pltpu.SemaphoreType.DMA,
        ],
    )
    return f(emb, ids32)


def kernel(batch_gnn_embed, gather_idx, num_graph):
    ids32 = gather_idx.astype(jnp.int32)
    out = _seg_max(batch_gnn_embed, ids32)
    return out[:10000]


# speedup vs baseline: 1.9803x; 1.3318x over previous
import jax
import jax.numpy as jnp
from jax import lax
from jax.experimental import pallas as pl
from jax.experimental.pallas import tpu as pltpu
from jax.experimental.pallas import tpu_sc as plsc

NC = 2
NS = 16
L = 16
NW = NC * NS
G = 320
OUT_ROWS = NW * G
D = 128
NV = D // L
B = 256
N = 320000
NBLK = N // B
BS_STEPS = 11
NEG_INF = float("-inf")


def _seg_max_body(emb_hbm, ids_hbm, out_hbm,
                  acc, rowbuf0, rowbuf1, idbuf0, idbuf1, probe_a, probe_b,
                  sem_a, sem_b, semb0, semb1):
    i32 = jnp.int32
    cid = lax.axis_index("c")
    sid = lax.axis_index("s")
    wid = (sid * NC + cid).astype(i32)
    lo = wid * i32(G)

    ninf = jnp.full((L,), NEG_INF, jnp.float32)

    def init_row(j, carry):
        for v in range(NV):
            acc[j, pl.ds(v * L, L)] = ninf
        return carry

    lax.fori_loop(i32(0), i32(G + 1), init_row, i32(0))

    t1 = lo
    t2 = lo + G

    def bs_step(_, st):
        a1, b1, a2, b2 = st
        m1 = jnp.minimum((a1 + b1) >> 1, i32(NBLK - 1))
        m2 = jnp.minimum((a2 + b2) >> 1, i32(NBLK - 1))
        cp1 = pltpu.async_copy(ids_hbm.at[pl.ds(m1 * B, L)], probe_a, sem_a)
        cp2 = pltpu.async_copy(ids_hbm.at[pl.ds(m2 * B, L)], probe_b, sem_b)
        cp1.wait()
        cp2.wait()
        v1 = probe_a[pl.ds(0, L)][0]
        v2 = probe_b[pl.ds(0, L)][0]
        a1n = jnp.where(a1 < b1, jnp.where(v1 < t1, m1 + 1, a1), a1)
        b1n = jnp.where(a1 < b1, jnp.where(v1 < t1, b1, m1), b1)
        a2n = jnp.where(a2 < b2, jnp.where(v2 < t2, m2 + 1, a2), a2)
        b2n = jnp.where(a2 < b2, jnp.where(v2 < t2, b2, m2), b2)
        return (a1n, b1n, a2n, b2n)

    kk, _, ke, _ = lax.fori_loop(
        i32(0), i32(BS_STEPS), bs_step,
        (i32(0), i32(NBLK), i32(0), i32(NBLK)))
    k0 = jnp.maximum(kk - i32(1), i32(0))
    nb = jnp.maximum(ke - k0, i32(0))

    def issue(bi, rbuf, ibuf, sem):
        r = (k0 + bi) * i32(B)
        pltpu.async_copy(ids_hbm.at[pl.ds(r, B)], ibuf, sem)
        pltpu.async_copy(emb_hbm.at[pl.ds(r, B), :], rbuf, sem)

    def wait_buf(rbuf, ibuf, sem):
        pltpu.make_async_copy(ids_hbm.at[pl.ds(0, B)], ibuf, sem).wait()
        pltpu.make_async_copy(emb_hbm.at[pl.ds(0, B), :], rbuf, sem).wait()

    def process(rbuf, ibuf):
        def group_body(g, c2):
            idvec = ibuf[pl.ds(g * i32(L), L)]
            slot = idvec - lo
            inr = (slot >= i32(0)) & (slot < i32(G))
            svec = jnp.where(inr, slot, i32(G))
            for j in range(L):
                s = svec[j]
                for v in range(NV):
                    sl = pl.ds(v * L, L)
                    acc[s, sl] = jnp.maximum(
                        acc[s, sl], rbuf[g * i32(L) + i32(j), sl])
            return c2

        lax.fori_loop(i32(0), i32(B // L), group_body, i32(0))

    @pl.when(nb > i32(0))
    def _():
        issue(i32(0), rowbuf0, idbuf0, semb0)

    nt = (nb + i32(1)) >> 1

    def pair_body(t, carry):
        b1 = t * i32(2) + i32(1)

        @pl.when(b1 < nb)
        def _():
            issue(b1, rowbuf1, idbuf1, semb1)

        wait_buf(rowbuf0, idbuf0, semb0)
        process(rowbuf0, idbuf0)

        @pl.when(b1 + i32(1) < nb)
        def _():
            issue(b1 + i32(1), rowbuf0, idbuf0, semb0)

        @pl.when(b1 < nb)
        def _():
            wait_buf(rowbuf1, idbuf1, semb1)
            process(rowbuf1, idbuf1)

        return carry

    lax.fori_loop(i32(0), nt, pair_body, i32(0))

    zero = jnp.zeros((L,), jnp.float32)

    def fix_row(j, carry):
        for v in range(NV):
            sl = pl.ds(v * L, L)
            x = acc[j, sl]
            acc[j, sl] = jnp.where(x == NEG_INF, zero, x)
        return carry

    lax.fori_loop(i32(0), i32(G), fix_row, i32(0))
    pltpu.sync_copy(acc.at[pl.ds(0, G)], out_hbm.at[pl.ds(lo, G)])


@jax.jit
def _seg_max(emb, ids32):
    mesh = plsc.VectorSubcoreMesh(core_axis_name="c", subcore_axis_name="s")
    f = pl.kernel(
        _seg_max_body,
        out_type=jax.ShapeDtypeStruct((OUT_ROWS, D), jnp.float32),
        mesh=mesh,
        scratch_types=[
            pltpu.VMEM((G + 1, D), jnp.float32),
            pltpu.VMEM((B, D), jnp.float32),
            pltpu.VMEM((B, D), jnp.float32),
            pltpu.VMEM((B,), jnp.int32),
            pltpu.VMEM((B,), jnp.int32),
            pltpu.VMEM((L,), jnp.int32),
            pltpu.VMEM((L,), jnp.int32),
            pltpu.SemaphoreType.DMA,
            pltpu.SemaphoreType.DMA,
            pltpu.SemaphoreType.DMA,
            pltpu.SemaphoreType.DMA,
        ],
    )
    return f(emb, ids32)


def kernel(batch_gnn_embed, gather_idx, num_graph):
    ids32 = gather_idx.astype(jnp.int32)
    out = _seg_max(batch_gnn_embed, ids32)
    return out[:10000]
